# baseline (speedup 1.0000x reference)
import functools

import jax
import jax.numpy as jnp
from jax import lax
from jax.experimental import pallas as pl
from jax.experimental.pallas import tpu as pltpu
from jax.experimental.pallas import tpu_sc as plsc

NUM_CLS = 32
D_MODEL = 256
BATCH = 4096
HIST = 50

NC, NS = 2, 16
NW = NC * NS
ROWS = BATCH * HIST
R_PER_W = ROWS // NW
K_REP = 8
HALF = D_MODEL // 128
CHUNK = 64
NCHUNK = R_PER_W // CHUNK
NCHUNK_PAD = 104
PAIRS = NCHUNK // 2
TAB_ROWS = NUM_CLS * HALF


@functools.partial(
    pl.kernel,
    out_type=jax.ShapeDtypeStruct((ROWS * HALF, 128), jnp.float32),
    mesh=plsc.VectorSubcoreMesh(core_axis_name="c", subcore_axis_name="s"),
    scratch_types=[
        pltpu.VMEM((NCHUNK_PAD, CHUNK * HALF), jnp.int32),
        pltpu.VMEM((CHUNK * HALF, 128), jnp.float32),
        pltpu.VMEM((CHUNK * HALF, 128), jnp.float32),
        pltpu.SemaphoreType.DMA,
        pltpu.SemaphoreType.DMA,
        pltpu.SemaphoreType.DMA,
        pltpu.SemaphoreType.DMA,
    ],
)
def _embed_sc(table_hbm, idx_hbm, out_hbm, idx_v, buf_a, buf_b,
              gsem_a, gsem_b, wsem_a, wsem_b):
    wid = lax.axis_index("s") * NC + lax.axis_index("c")
    base = wid * R_PER_W * HALF

    pltpu.sync_copy(idx_hbm.at[pl.ds(wid * NCHUNK_PAD, NCHUNK_PAD)], idx_v)

    def gather(c, buf, sem):
        pltpu.async_copy(table_hbm.at[idx_v.at[c]], buf, sem)

    def wait_gather(c, buf, sem):
        pltpu.make_async_copy(table_hbm.at[idx_v.at[c]], buf, sem).wait()

    def write(c, buf, sem):
        pltpu.async_copy(
            buf, out_hbm.at[pl.ds(base + c * CHUNK * HALF, CHUNK * HALF)], sem)

    def wait_write(c, buf, sem):
        pltpu.make_async_copy(
            buf, out_hbm.at[pl.ds(base + c * CHUNK * HALF, CHUNK * HALF)],
            sem).wait()

    gather(0, buf_a, gsem_a)

    def pair(i):
        c0 = i * 2
        gather(c0 + 1, buf_b, gsem_b)
        wait_gather(c0, buf_a, gsem_a)
        write(c0, buf_a, wsem_a)
        @pl.when(i < PAIRS - 1)
        def _():
            wait_write(c0, buf_a, wsem_a)
            gather(c0 + 2, buf_a, gsem_a)
        wait_gather(c0 + 1, buf_b, gsem_b)
        write(c0 + 1, buf_b, wsem_b)
        @pl.when(i < PAIRS - 1)
        def _():
            wait_write(c0 + 1, buf_b, wsem_b)

    pl.loop(0, PAIRS)(pair)
    wait_write(NCHUNK - 2, buf_a, wsem_a)
    wait_write(NCHUNK - 1, buf_b, wsem_b)


def kernel(data, table):
    flat = data.reshape(-1)
    i = jnp.arange(ROWS, dtype=jnp.int32)
    rep_id = (i // R_PER_W) * K_REP + (i % K_REP)
    hbase = rep_id * TAB_ROWS + HALF * flat
    j = jnp.stack([hbase, hbase + 1], axis=-1).reshape(NW, NCHUNK, CHUNK * HALF)
    j = jnp.pad(j, ((0, 0), (0, NCHUNK_PAD - NCHUNK), (0, 0)))
    j = j.reshape(NW * NCHUNK_PAD, CHUNK * HALF)
    rep = jnp.tile(table.reshape(TAB_ROWS, 128), (NW * K_REP, 1))
    out = _embed_sc(rep, j)
    return out.reshape(BATCH, HIST, D_MODEL)


# speedup vs baseline: 1.0029x; 1.0029x over previous
import functools

import jax
import jax.numpy as jnp
from jax import lax
from jax.experimental import pallas as pl
from jax.experimental.pallas import tpu as pltpu
from jax.experimental.pallas import tpu_sc as plsc

NUM_CLS = 32
D_MODEL = 256
BATCH = 4096
HIST = 50

NC, NS = 2, 16
NW = NC * NS
ROWS = BATCH * HIST
R_PER_W = ROWS // NW
K_REP = 8
CHUNK = 128
NCHUNK = R_PER_W // CHUNK
PAIRS = NCHUNK // 2

BB = 8
R_PER_BB = BB * HIST


@functools.partial(
    pl.kernel,
    out_type=jax.ShapeDtypeStruct((ROWS, D_MODEL), jnp.float32),
    mesh=plsc.VectorSubcoreMesh(core_axis_name="c", subcore_axis_name="s"),
    scratch_types=[
        pltpu.VMEM((NCHUNK, CHUNK), jnp.int32),
        pltpu.VMEM((CHUNK, D_MODEL), jnp.float32),
        pltpu.VMEM((CHUNK, D_MODEL), jnp.float32),
        pltpu.SemaphoreType.DMA,
        pltpu.SemaphoreType.DMA,
        pltpu.SemaphoreType.DMA,
        pltpu.SemaphoreType.DMA,
    ],
)
def _embed_sc(table_hbm, idx_hbm, out_hbm, idx_v, buf_a, buf_b,
              gsem_a, gsem_b, wsem_a, wsem_b):
    wid = lax.axis_index("s") * NC + lax.axis_index("c")
    base = wid * R_PER_W

    pltpu.sync_copy(idx_hbm.at[wid], idx_v)

    def gather(c, buf, sem):
        pltpu.async_copy(table_hbm.at[idx_v.at[c]], buf, sem)

    def wait_gather(c, buf, sem):
        pltpu.make_async_copy(table_hbm.at[idx_v.at[c]], buf, sem).wait()

    def write(c, buf, sem):
        pltpu.async_copy(buf, out_hbm.at[pl.ds(base + c * CHUNK, CHUNK)], sem)

    def wait_write(c, buf, sem):
        pltpu.make_async_copy(
            buf, out_hbm.at[pl.ds(base + c * CHUNK, CHUNK)], sem).wait()

    gather(0, buf_a, gsem_a)

    def pair(i):
        c0 = i * 2
        gather(c0 + 1, buf_b, gsem_b)
        wait_gather(c0, buf_a, gsem_a)
        write(c0, buf_a, wsem_a)
        @pl.when(i < PAIRS - 1)
        def _():
            wait_write(c0, buf_a, wsem_a)
            gather(c0 + 2, buf_a, gsem_a)
        wait_gather(c0 + 1, buf_b, gsem_b)
        write(c0 + 1, buf_b, wsem_b)
        @pl.when(i < PAIRS - 1)
        def _():
            wait_write(c0 + 1, buf_b, wsem_b)

    pl.loop(0, PAIRS)(pair)
    wait_write(NCHUNK - 2, buf_a, wsem_a)
    wait_write(NCHUNK - 1, buf_b, wsem_b)


def _finish_body(rows_hbm, out_ref, scr_a, scr_b, sem_a, sem_b):
    b = pl.program_id(0)
    nblk = pl.num_programs(0)

    def load(blk, scr, sem):
        return pltpu.make_async_copy(
            rows_hbm.at[pl.ds(blk * R_PER_BB, R_PER_BB)], scr, sem)

    @pl.when(b == 0)
    def _():
        load(0, scr_a, sem_a).start()

    @pl.when(b + 1 < nblk)
    def _():
        @pl.when(lax.rem(b, 2) == 0)
        def _():
            load(b + 1, scr_b, sem_b).start()
        @pl.when(lax.rem(b, 2) == 1)
        def _():
            load(b + 1, scr_a, sem_a).start()

    @pl.when(lax.rem(b, 2) == 0)
    def _():
        load(b, scr_a, sem_a).wait()
        out_ref[...] = scr_a[...].reshape(BB, HIST, D_MODEL)

    @pl.when(lax.rem(b, 2) == 1)
    def _():
        load(b, scr_b, sem_b).wait()
        out_ref[...] = scr_b[...].reshape(BB, HIST, D_MODEL)


_finish = pl.pallas_call(
    _finish_body,
    grid=(BATCH // BB,),
    in_specs=[pl.BlockSpec(memory_space=pl.ANY)],
    out_specs=pl.BlockSpec((BB, HIST, D_MODEL), lambda b: (b, 0, 0)),
    out_shape=jax.ShapeDtypeStruct((BATCH, HIST, D_MODEL), jnp.float32),
    scratch_shapes=[
        pltpu.VMEM((R_PER_BB, D_MODEL), jnp.float32),
        pltpu.VMEM((R_PER_BB, D_MODEL), jnp.float32),
        pltpu.SemaphoreType.DMA,
        pltpu.SemaphoreType.DMA,
    ],
)


def kernel(data, table):
    flat = data.reshape(-1)
    i = jnp.arange(ROWS, dtype=jnp.int32)
    offs = (i // R_PER_W) * K_REP + (i % K_REP)
    idx = (flat + NUM_CLS * offs).reshape(NW, NCHUNK, CHUNK)
    rep = jnp.tile(table, (NW * K_REP, 1))
    rows = _embed_sc(rep, idx)
    return _finish(rows)


# speedup vs baseline: 1.0053x; 1.0023x over previous
import functools

import jax
import jax.numpy as jnp
from jax import lax
from jax.experimental import pallas as pl
from jax.experimental.pallas import tpu as pltpu
from jax.experimental.pallas import tpu_sc as plsc

NUM_CLS = 32
D_MODEL = 256
BATCH = 4096
HIST = 50

NC, NS = 2, 16
NW = NC * NS
ROWS = BATCH * HIST
R_PER_W = ROWS // NW
K_REP = 8
CHUNK = 128
NCHUNK = R_PER_W // CHUNK
PAIRS = NCHUNK // 2

BB = 8
R_PER_BB = BB * HIST


@functools.partial(
    pl.kernel,
    out_type=jax.ShapeDtypeStruct((ROWS, D_MODEL), jnp.float32),
    mesh=plsc.VectorSubcoreMesh(core_axis_name="c", subcore_axis_name="s"),
    scratch_types=[
        pltpu.VMEM((NCHUNK, CHUNK), jnp.int32),
        pltpu.VMEM((CHUNK, D_MODEL), jnp.float32),
        pltpu.VMEM((CHUNK, D_MODEL), jnp.float32),
        pltpu.SemaphoreType.DMA,
        pltpu.SemaphoreType.DMA,
        pltpu.SemaphoreType.DMA,
        pltpu.SemaphoreType.DMA,
    ],
)
def _embed_sc(table_hbm, idx_hbm, out_hbm, idx_v, buf_a, buf_b,
              gsem_a, gsem_b, wsem_a, wsem_b):
    wid = lax.axis_index("s") * NC + lax.axis_index("c")
    base = wid * R_PER_W

    pltpu.sync_copy(idx_hbm.at[wid], idx_v)

    def gather(c, buf, sem):
        pltpu.async_copy(table_hbm.at[idx_v.at[c]], buf, sem)

    def wait_gather(c, buf, sem):
        pltpu.make_async_copy(table_hbm.at[idx_v.at[c]], buf, sem).wait()

    def write(c, buf, sem):
        pltpu.async_copy(buf, out_hbm.at[pl.ds(base + c * CHUNK, CHUNK)], sem)

    def wait_write(c, buf, sem):
        pltpu.make_async_copy(
            buf, out_hbm.at[pl.ds(base + c * CHUNK, CHUNK)], sem).wait()

    gather(0, buf_a, gsem_a)

    def pair(i):
        c0 = i * 2
        gather(c0 + 1, buf_b, gsem_b)
        wait_gather(c0, buf_a, gsem_a)
        write(c0, buf_a, wsem_a)
        @pl.when(i < PAIRS - 1)
        def _():
            wait_write(c0, buf_a, wsem_a)
            gather(c0 + 2, buf_a, gsem_a)
        wait_gather(c0 + 1, buf_b, gsem_b)
        write(c0 + 1, buf_b, wsem_b)
        @pl.when(i < PAIRS - 1)
        def _():
            wait_write(c0 + 1, buf_b, wsem_b)

    pl.loop(0, PAIRS)(pair)
    wait_write(NCHUNK - 2, buf_a, wsem_a)
    wait_write(NCHUNK - 1, buf_b, wsem_b)


def _finish_body(rows_hbm, out_hbm, in_a, in_b, ob_a, ob_b,
                 isem_a, isem_b, osem_a, osem_b):
    b = pl.program_id(0)
    nblk = pl.num_programs(0)
    ins = (in_a, in_b)
    isems = (isem_a, isem_b)
    obs = (ob_a, ob_b)
    osems = (osem_a, osem_b)

    def load(blk, par):
        return pltpu.make_async_copy(
            rows_hbm.at[pl.ds(blk * R_PER_BB, R_PER_BB)], ins[par], isems[par])

    def store(blk, par):
        return pltpu.make_async_copy(
            obs[par], out_hbm.at[pl.ds(blk * BB, BB)], osems[par])

    @pl.when(b == 0)
    def _():
        load(0, 0).start()

    for par in (0, 1):
        @pl.when(lax.rem(b, 2) == par)
        def _(par=par):
            @pl.when(b >= 2)
            def _():
                store(b - 2, par).wait()
            @pl.when(b + 1 < nblk)
            def _():
                load(b + 1, 1 - par).start()
            load(b, par).wait()
            obs[par][...] = ins[par][...].reshape(BB, HIST, D_MODEL)
            store(b, par).start()
            @pl.when(b == nblk - 1)
            def _():
                store(b - 1, 1 - par).wait()
                store(b, par).wait()


_finish = pl.pallas_call(
    _finish_body,
    grid=(BATCH // BB,),
    in_specs=[pl.BlockSpec(memory_space=pl.ANY)],
    out_specs=pl.BlockSpec(memory_space=pl.ANY),
    out_shape=jax.ShapeDtypeStruct((BATCH, HIST, D_MODEL), jnp.float32),
    scratch_shapes=[
        pltpu.VMEM((R_PER_BB, D_MODEL), jnp.float32),
        pltpu.VMEM((R_PER_BB, D_MODEL), jnp.float32),
        pltpu.VMEM((BB, HIST, D_MODEL), jnp.float32),
        pltpu.VMEM((BB, HIST, D_MODEL), jnp.float32),
        pltpu.SemaphoreType.DMA,
        pltpu.SemaphoreType.DMA,
        pltpu.SemaphoreType.DMA,
        pltpu.SemaphoreType.DMA,
    ],
)


def kernel(data, table):
    flat = data.reshape(-1)
    i = jnp.arange(ROWS, dtype=jnp.int32)
    offs = (i // R_PER_W) * K_REP + (i % K_REP)
    idx = (flat + NUM_CLS * offs).reshape(NW, NCHUNK, CHUNK)
    rep = jnp.tile(table, (NW * K_REP, 1))
    rows = _embed_sc(rep, idx)
    return _finish(rows)


# speedup vs baseline: 1.4654x; 1.4577x over previous
import functools

import jax
import jax.numpy as jnp
from jax import lax
from jax.experimental import pallas as pl
from jax.experimental.pallas import tpu as pltpu
from jax.experimental.pallas import tpu_sc as plsc

NUM_CLS = 32
D_MODEL = 256
BATCH = 4096
HIST = 50

NC, NS = 2, 16
NW = NC * NS
ROWS = BATCH * HIST
R_PER_W = ROWS // NW
K_REP = 8
CHUNK = 128
NCHUNK = R_PER_W // CHUNK
PAIRS = NCHUNK // 2

FB = 512
NFB = BATCH // FB
NBLK = HIST * NFB


@functools.partial(
    pl.kernel,
    out_type=jax.ShapeDtypeStruct((ROWS, D_MODEL), jnp.float32),
    mesh=plsc.VectorSubcoreMesh(core_axis_name="c", subcore_axis_name="s"),
    scratch_types=[
        pltpu.VMEM((NCHUNK, CHUNK), jnp.int32),
        pltpu.VMEM((CHUNK, D_MODEL), jnp.float32),
        pltpu.VMEM((CHUNK, D_MODEL), jnp.float32),
        pltpu.SemaphoreType.DMA,
        pltpu.SemaphoreType.DMA,
        pltpu.SemaphoreType.DMA,
        pltpu.SemaphoreType.DMA,
    ],
)
def _embed_sc(table_hbm, idx_hbm, out_hbm, idx_v, buf_a, buf_b,
              gsem_a, gsem_b, wsem_a, wsem_b):
    wid = lax.axis_index("s") * NC + lax.axis_index("c")
    base = wid * R_PER_W

    pltpu.sync_copy(idx_hbm.at[wid], idx_v)

    def gather(c, buf, sem):
        pltpu.async_copy(table_hbm.at[idx_v.at[c]], buf, sem)

    def wait_gather(c, buf, sem):
        pltpu.make_async_copy(table_hbm.at[idx_v.at[c]], buf, sem).wait()

    def write(c, buf, sem):
        pltpu.async_copy(buf, out_hbm.at[pl.ds(base + c * CHUNK, CHUNK)], sem)

    def wait_write(c, buf, sem):
        pltpu.make_async_copy(
            buf, out_hbm.at[pl.ds(base + c * CHUNK, CHUNK)], sem).wait()

    gather(0, buf_a, gsem_a)

    def pair(i):
        c0 = i * 2
        gather(c0 + 1, buf_b, gsem_b)
        wait_gather(c0, buf_a, gsem_a)
        write(c0, buf_a, wsem_a)
        @pl.when(i < PAIRS - 1)
        def _():
            wait_write(c0, buf_a, wsem_a)
            gather(c0 + 2, buf_a, gsem_a)
        wait_gather(c0 + 1, buf_b, gsem_b)
        write(c0 + 1, buf_b, wsem_b)
        @pl.when(i < PAIRS - 1)
        def _():
            wait_write(c0 + 1, buf_b, wsem_b)

    pl.loop(0, PAIRS)(pair)
    wait_write(NCHUNK - 2, buf_a, wsem_a)
    wait_write(NCHUNK - 1, buf_b, wsem_b)


def _finish_body(rows_hbm, out_ref, in_a, in_b, isem_a, isem_b):
    b = pl.program_id(0)
    ins = (in_a, in_b)
    isems = (isem_a, isem_b)

    def load(blk, par):
        return pltpu.make_async_copy(
            rows_hbm.at[pl.ds(blk * FB, FB)], ins[par], isems[par])

    @pl.when(b == 0)
    def _():
        load(0, 0).start()

    @pl.when(b + 1 < NBLK)
    def _():
        for par in (0, 1):
            @pl.when(lax.rem(b + 1, 2) == par)
            def _(par=par):
                load(b + 1, par).start()

    for par in (0, 1):
        @pl.when(lax.rem(b, 2) == par)
        def _(par=par):
            load(b, par).wait()
            out_ref[...] = ins[par][...].reshape(1, FB, D_MODEL)


_finish = pl.pallas_call(
    _finish_body,
    grid=(NBLK,),
    in_specs=[pl.BlockSpec(memory_space=pl.ANY)],
    out_specs=pl.BlockSpec((1, FB, D_MODEL), lambda b: (b // NFB, b % NFB, 0)),
    out_shape=jax.ShapeDtypeStruct((HIST, BATCH, D_MODEL), jnp.float32),
    scratch_shapes=[
        pltpu.VMEM((FB, D_MODEL), jnp.float32),
        pltpu.VMEM((FB, D_MODEL), jnp.float32),
        pltpu.SemaphoreType.DMA,
        pltpu.SemaphoreType.DMA,
    ],
)


def kernel(data, table):
    flat = data.T.reshape(-1)
    i = jnp.arange(ROWS, dtype=jnp.int32)
    offs = (i // R_PER_W) * K_REP + (i % K_REP)
    idx = (flat + NUM_CLS * offs).reshape(NW, NCHUNK, CHUNK)
    rep = jnp.tile(table, (NW * K_REP, 1))
    rows = _embed_sc(rep, idx)
    return _finish(rows).transpose(1, 0, 2)
